# Initial kernel scaffold; baseline (speedup 1.0000x reference)
import jax
import jax.numpy as jnp
from jax.experimental import pallas as pl


def kernel(Struct, Label):
    raise NotImplementedError("write your pallas kernel here")


# speedup vs baseline: 40.2401x; 40.2401x over previous
import math

import jax
import jax.numpy as jnp
from jax import lax
from jax.experimental import pallas as pl
from jax.experimental.pallas import tpu as pltpu

_K = 4
_STD = 5.0
_BLK = 256


def _body(struct_full_ref, s_blk_ref, lrow_ref, lcol_ref, out_ref):
    i = pl.program_id(0)
    B = lrow_ref.shape[1]
    R = lcol_ref.shape[0]

    lab_row = lrow_ref[:, :]
    lab_col = lcol_ref[:, :]
    dist = jnp.abs(lab_row - lab_col)
    iota = lax.broadcasted_iota(jnp.int32, (R, B), 1)

    inv2s2 = 1.0 / (2.0 * _STD * _STD)
    wacc = jnp.zeros((R, B), jnp.float32)
    for _ in range(_K):
        m = jnp.min(dist, axis=1, keepdims=True)
        cand = jnp.where(dist == m, iota, B)
        jdx = jnp.min(cand, axis=1, keepdims=True)
        sel = iota == jdx
        w = jnp.exp(-(m * m) * inv2s2)
        wacc = wacc + jnp.where(sel, w, 0.0)
        dist = jnp.where(sel, jnp.inf, dist)

    wsum = jnp.sum(wacc, axis=1, keepdims=True)
    wmat = wacc / wsum

    mean = jnp.dot(wmat, struct_full_ref[:, :],
                   preferred_element_type=jnp.float32)
    s = s_blk_ref[:, :]
    n1 = jnp.sqrt(jnp.sum(s * s, axis=1, keepdims=True))
    n2 = jnp.sqrt(jnp.sum(mean * mean, axis=1, keepdims=True))
    sm = (s / (1e-10 + n1)) * (mean / (1e-10 + n2))
    partial = jnp.sum(sm)

    @pl.when(i == 0)
    def _():
        out_ref[0, 0] = 0.0

    out_ref[0, 0] += partial

    @pl.when(i == pl.num_programs(0) - 1)
    def _():
        out_ref[0, 0] = 1.0 - out_ref[0, 0] / B


def kernel(Struct, Label):
    B, D = Struct.shape
    lrow = Label.reshape(1, B)
    lcol = Label.reshape(B, 1)
    out = pl.pallas_call(
        _body,
        grid=(B // _BLK,),
        in_specs=[
            pl.BlockSpec((B, D), lambda i: (0, 0)),
            pl.BlockSpec((_BLK, D), lambda i: (i, 0)),
            pl.BlockSpec((1, B), lambda i: (0, 0)),
            pl.BlockSpec((_BLK, 1), lambda i: (i, 0)),
        ],
        out_specs=pl.BlockSpec(memory_space=pltpu.SMEM),
        out_shape=jax.ShapeDtypeStruct((1, 1), jnp.float32),
    )(Struct, Struct, lrow, lcol)
    return out[0, 0]
